# Initial kernel scaffold; baseline (speedup 1.0000x reference)
import jax
import jax.numpy as jnp
from jax.experimental import pallas as pl


def kernel(token_ids, weight):
    raise NotImplementedError("write your pallas kernel here")


# speedup vs baseline: 1.6867x; 1.6867x over previous
import functools

import jax
import jax.numpy as jnp
from jax import lax
from jax.experimental import pallas as pl
from jax.experimental.pallas import tpu as pltpu
from jax.experimental.pallas import tpu_sc as plsc

NUM_CORES = 2
NUM_SUBCORES = 16
NUM_TILES = NUM_CORES * NUM_SUBCORES
CHUNK = 128


@functools.lru_cache(maxsize=None)
def _build(B, D, n_chunks):
    rows_per_tile = n_chunks * CHUNK
    mesh = plsc.VectorSubcoreMesh(core_axis_name="c", subcore_axis_name="s")

    def body(tok_ref, table_ref, out_ref, idx_v, rows_v, sem):
        wid = lax.axis_index("s") * NUM_CORES + lax.axis_index("c")
        pltpu.sync_copy(tok_ref.at[pl.ds(wid * n_chunks, n_chunks)], idx_v)
        base = wid * rows_per_tile

        @pl.loop(0, n_chunks)
        def _step(j):
            pltpu.async_copy(table_ref.at[idx_v.at[j]], rows_v, sem).wait()
            pltpu.sync_copy(rows_v, out_ref.at[pl.ds(base + j * CHUNK, CHUNK)])

    return pl.kernel(
        body,
        out_type=jax.ShapeDtypeStruct((B, D), jnp.float32),
        mesh=mesh,
        scratch_types=[
            pltpu.VMEM((n_chunks, CHUNK), jnp.int32),
            pltpu.VMEM((CHUNK, D), jnp.float32),
            pltpu.SemaphoreType.DMA,
        ],
        compiler_params=pltpu.CompilerParams(use_tc_tiling_on_sc=False),
    )


def kernel(token_ids, weight):
    Bt, S = token_ids.shape
    V, D = weight.shape
    B = Bt * S
    n_chunks = B // (NUM_TILES * CHUNK)
    tok = token_ids.reshape(B // CHUNK, CHUNK).astype(jnp.int32)
    out = _build(B, D, n_chunks)(tok, weight)
    return out.reshape(Bt, S, D)


# speedup vs baseline: 1.8749x; 1.1115x over previous
import functools

import jax
import jax.numpy as jnp
from jax import lax
from jax.experimental import pallas as pl
from jax.experimental.pallas import tpu as pltpu
from jax.experimental.pallas import tpu_sc as plsc

NUM_CORES = 2
NUM_SUBCORES = 16
NUM_TILES = NUM_CORES * NUM_SUBCORES
CHUNK = 128
GROUP_CHUNKS = 5


@functools.lru_cache(maxsize=None)
def _build(B, D, n_chunks):
    assert n_chunks % (2 * GROUP_CHUNKS) == 0
    n_groups = n_chunks // GROUP_CHUNKS
    mesh = plsc.VectorSubcoreMesh(core_axis_name="c", subcore_axis_name="s")

    def body(tok_ref, table_ref, out_ref, idx_v, rows0, rows1, g0, g1, o0, o1):
        wid = lax.axis_index("s") * NUM_CORES + lax.axis_index("c")
        pltpu.sync_copy(tok_ref.at[pl.ds(wid * n_chunks, n_chunks)], idx_v)
        chunk_base = wid * n_chunks
        rows = (rows0, rows1)
        gsem = (g0, g1)
        osem = (o0, o1)

        def fire_gathers(g, buf):
            for c in range(GROUP_CHUNKS):
                pltpu.async_copy(
                    table_ref.at[idx_v.at[g * GROUP_CHUNKS + c]],
                    rows[buf].at[c], gsem[buf])

        def wait_gathers(buf):
            for c in range(GROUP_CHUNKS):
                pltpu.make_async_copy(
                    table_ref.at[idx_v.at[c]], rows[buf].at[c],
                    gsem[buf]).wait()

        def fire_out(g, buf):
            pltpu.async_copy(
                rows[buf],
                out_ref.at[pl.ds(chunk_base + g * GROUP_CHUNKS, GROUP_CHUNKS)],
                osem[buf])

        def wait_out(g, buf):
            pltpu.make_async_copy(
                rows[buf],
                out_ref.at[pl.ds(chunk_base + g * GROUP_CHUNKS, GROUP_CHUNKS)],
                osem[buf]).wait()

        fire_gathers(0, 0)
        fire_gathers(1, 1)
        wait_gathers(0)
        fire_out(0, 0)

        @pl.loop(1, n_groups - 1, step=2)
        def _pair(g):
            wait_out(g - 1, 0)
            fire_gathers(g + 1, 0)
            wait_gathers(1)
            fire_out(g, 1)

            wait_out(g, 1)
            fire_gathers(g + 2, 1)
            wait_gathers(0)
            fire_out(g + 1, 0)

        g_last = n_groups - 1
        wait_gathers(1)
        fire_out(g_last, 1)
        wait_out(g_last - 1, 0)
        wait_out(g_last, 1)

    return pl.kernel(
        body,
        out_type=jax.ShapeDtypeStruct((B // CHUNK, CHUNK, D), jnp.float32),
        mesh=mesh,
        scratch_types=[
            pltpu.VMEM((n_chunks, CHUNK), jnp.int32),
            pltpu.VMEM((GROUP_CHUNKS, CHUNK, D), jnp.float32),
            pltpu.VMEM((GROUP_CHUNKS, CHUNK, D), jnp.float32),
            pltpu.SemaphoreType.DMA,
            pltpu.SemaphoreType.DMA,
            pltpu.SemaphoreType.DMA,
            pltpu.SemaphoreType.DMA,
        ],
        compiler_params=pltpu.CompilerParams(use_tc_tiling_on_sc=False),
    )


def kernel(token_ids, weight):
    Bt, S = token_ids.shape
    V, D = weight.shape
    B = Bt * S
    n_chunks = B // (NUM_TILES * CHUNK)
    tok = token_ids.reshape(B // CHUNK, CHUNK).astype(jnp.int32)
    out = _build(B, D, n_chunks)(tok, weight)
    return out.reshape(Bt, S, D)
